# Initial kernel scaffold; baseline (speedup 1.0000x reference)
import jax
import jax.numpy as jnp
from jax.experimental import pallas as pl


def kernel(x, index_scores, training_step):
    raise NotImplementedError("write your pallas kernel here")


# speedup vs baseline: 14.5920x; 14.5920x over previous
import functools

import jax
import jax.numpy as jnp
from jax.experimental import pallas as pl
from jax.experimental.pallas import tpu as pltpu

_START_K = 16
_END_K = 256
_MAX_STEPS = 1000
_STEP_CONST = 500

_INT_MIN = -(2**31)


def _mask_kernel(scores_ref, out_ref, *, k_static, tq, s):
    blk_q = pl.program_id(1)
    scores = scores_ref[0]
    bits = jax.lax.bitcast_convert_type(scores, jnp.int32)
    key = bits ^ (jax.lax.shift_right_arithmetic(bits, 31) & jnp.int32(0x7FFFFFFF))
    q = blk_q * tq + jax.lax.broadcasted_iota(jnp.int32, (tq, 1), 0)
    j = jax.lax.broadcasted_iota(jnp.int32, (tq, s), 1)
    key = jnp.where(j <= q, key, jnp.int32(_INT_MIN))
    kq = jnp.minimum(jnp.int32(k_static), q + 1)

    cnt = jnp.sum((key >= 0).astype(jnp.int32), axis=1, keepdims=True)
    res = jnp.where(cnt >= kq, jnp.int32(0), jnp.int32(_INT_MIN))
    for bit in range(30, -1, -1):
        cand = res | jnp.int32(1 << bit)
        cnt = jnp.sum((key >= cand).astype(jnp.int32), axis=1, keepdims=True)
        res = jnp.where(cnt >= kq, cand, res)

    gt = key > res
    cnt_gt = jnp.sum(gt.astype(jnp.int32), axis=1, keepdims=True)
    need = kq - cnt_gt
    eq = key == res
    needm1 = need - 1
    resc = jnp.zeros_like(kq)
    for bit in range((s - 1).bit_length() - 1, -1, -1):
        candc = resc | jnp.int32(1 << bit)
        cnte = jnp.sum((eq & (j < candc)).astype(jnp.int32), axis=1, keepdims=True)
        resc = jnp.where(cnte <= needm1, candc, resc)
    sel = eq & (j <= resc) & (need > 0)
    out_ref[0] = gt | sel


def kernel(x, index_scores, training_step):
    B, S, _ = index_scores.shape
    progress_static = min(1.0, _STEP_CONST / _MAX_STEPS)
    k_static = min(int(_START_K + (_END_K - _START_K) * progress_static), S)
    TQ = 256
    mask = pl.pallas_call(
        functools.partial(_mask_kernel, k_static=k_static, tq=TQ, s=S),
        grid=(B, S // TQ),
        in_specs=[pl.BlockSpec((1, TQ, S), lambda b, i: (b, i, 0))],
        out_specs=pl.BlockSpec((1, TQ, S), lambda b, i: (b, i, 0)),
        out_shape=jax.ShapeDtypeStruct((B, S, S), jnp.bool_),
        compiler_params=pltpu.CompilerParams(
            dimension_semantics=("parallel", "parallel")
        ),
    )(index_scores)

    progress_traced = jnp.minimum(1.0, training_step / _MAX_STEPS)
    k_traced = _START_K + (_END_K - _START_K) * progress_traced
    k_val = jnp.minimum(k_traced.astype(jnp.int32), S)
    k_values = jnp.broadcast_to(k_val, (B, S)).astype(jnp.int32)
    return (mask, k_values)


# speedup vs baseline: 18.2788x; 1.2527x over previous
import functools

import jax
import jax.numpy as jnp
from jax.experimental import pallas as pl
from jax.experimental.pallas import tpu as pltpu

_START_K = 16
_END_K = 256
_MAX_STEPS = 1000
_STEP_CONST = 500

_INT_MIN = -(2**31)


def _mask_kernel(scores_ref, out_ref, resc_ref, *, k_static, tq, s):
    blk_q = pl.program_id(1)
    scores = scores_ref[0]
    bits = jax.lax.bitcast_convert_type(scores, jnp.int32)
    key = bits ^ (jax.lax.shift_right_arithmetic(bits, 31) & jnp.int32(0x7FFFFFFF))
    q = blk_q * tq + jax.lax.broadcasted_iota(jnp.int32, (tq, 1), 0)
    j = jax.lax.broadcasted_iota(jnp.int32, (tq, s), 1)
    key = jnp.where(j <= q, key, jnp.int32(_INT_MIN))
    kq = jnp.minimum(jnp.int32(k_static), q + 1)

    cnt = jnp.sum((key >= 0).astype(jnp.int32), axis=1, keepdims=True)
    res = jnp.where(cnt >= kq, jnp.int32(0), jnp.int32(_INT_MIN))
    for bit in range(30, -1, -1):
        cand = res | jnp.int32(1 << bit)
        cnt = jnp.sum((key >= cand).astype(jnp.int32), axis=1, keepdims=True)
        res = jnp.where(cnt >= kq, cand, res)

    gt = key > res
    cnt_gt = jnp.sum(gt.astype(jnp.int32), axis=1, keepdims=True)
    need = kq - cnt_gt
    eq = key == res
    cnt_eq = jnp.sum(eq.astype(jnp.int32), axis=1, keepdims=True)

    resc_ref[...] = jnp.full((tq, 1), s - 1, jnp.int32)

    @pl.when(jnp.any(cnt_eq > need))
    def _tie_path():
        needm1 = need - 1
        resc = jnp.zeros_like(kq)
        for bit in range((s - 1).bit_length() - 1, -1, -1):
            candc = resc | jnp.int32(1 << bit)
            cnte = jnp.sum(
                (eq & (j < candc)).astype(jnp.int32), axis=1, keepdims=True
            )
            resc = jnp.where(cnte <= needm1, candc, resc)
        resc_ref[...] = resc

    sel = eq & (j <= resc_ref[...]) & (need > 0)
    out_ref[0] = gt | sel


def kernel(x, index_scores, training_step):
    B, S, _ = index_scores.shape
    progress_static = min(1.0, _STEP_CONST / _MAX_STEPS)
    k_static = min(int(_START_K + (_END_K - _START_K) * progress_static), S)
    TQ = 256
    mask = pl.pallas_call(
        functools.partial(_mask_kernel, k_static=k_static, tq=TQ, s=S),
        grid=(B, S // TQ),
        in_specs=[pl.BlockSpec((1, TQ, S), lambda b, i: (b, i, 0))],
        out_specs=pl.BlockSpec((1, TQ, S), lambda b, i: (b, i, 0)),
        out_shape=jax.ShapeDtypeStruct((B, S, S), jnp.bool_),
        scratch_shapes=[pltpu.VMEM((TQ, 1), jnp.int32)],
        compiler_params=pltpu.CompilerParams(
            dimension_semantics=("parallel", "parallel")
        ),
    )(index_scores)

    progress_traced = jnp.minimum(1.0, training_step / _MAX_STEPS)
    k_traced = _START_K + (_END_K - _START_K) * progress_traced
    k_val = jnp.minimum(k_traced.astype(jnp.int32), S)
    k_values = jnp.broadcast_to(k_val, (B, S)).astype(jnp.int32)
    return (mask, k_values)


# speedup vs baseline: 21.8171x; 1.1936x over previous
import functools

import jax
import jax.numpy as jnp
from jax.experimental import pallas as pl
from jax.experimental.pallas import tpu as pltpu

_START_K = 16
_END_K = 256
_MAX_STEPS = 1000
_STEP_CONST = 500

_INT_MIN = -(2**31)


def _mask_kernel(scores_ref, out_ref, resc_ref, *, k_static, tq, s):
    blk_q = pl.program_id(1)
    scores = scores_ref[0]
    bits = jax.lax.bitcast_convert_type(scores, jnp.int32)
    key = bits ^ (jax.lax.shift_right_arithmetic(bits, 31) & jnp.int32(0x7FFFFFFF))
    q = blk_q * tq + jax.lax.broadcasted_iota(jnp.int32, (tq, 1), 0)
    j = jax.lax.broadcasted_iota(jnp.int32, (tq, s), 1)
    key = jnp.where(j <= q, key, jnp.int32(_INT_MIN))
    kq = jnp.minimum(jnp.int32(k_static), q + 1)

    hi = jax.lax.shift_right_arithmetic(key, 16).astype(jnp.int16)
    lo = (key ^ jnp.int32(0x8000)).astype(jnp.int16)

    def cnt16(mask):
        a = mask.astype(jnp.int16)
        w = s
        while w > 128:
            w //= 2
            a = a[:, :w] + a[:, w:]
        return jnp.sum(a.astype(jnp.int32), axis=1, keepdims=True)

    cnt = cnt16(hi >= 0)
    res_hi = jnp.where(cnt >= kq, jnp.int32(0), jnp.int32(-(2**15)))
    for bit in range(14, -1, -1):
        cand = res_hi | jnp.int32(1 << bit)
        cnt = cnt16(hi >= cand.astype(jnp.int16))
        res_hi = jnp.where(cnt >= kq, cand, res_hi)

    res_hi16 = res_hi.astype(jnp.int16)
    ehi = hi == res_hi16
    cnt_hi_gt = cnt16(hi > res_hi16)

    kq_lo = kq - cnt_hi_gt
    cnt = cnt16(ehi & (lo >= 0))
    res_lo = jnp.where(cnt >= kq_lo, jnp.int32(0), jnp.int32(-(2**15)))
    for bit in range(14, -1, -1):
        cand = res_lo | jnp.int32(1 << bit)
        cnt = cnt16(ehi & (lo >= cand.astype(jnp.int16)))
        res_lo = jnp.where(cnt >= kq_lo, cand, res_lo)

    res = (res_hi << 16) | ((res_lo ^ jnp.int32(0x8000)) & jnp.int32(0xFFFF))
    gt = key > res
    cnt_gt = jnp.sum(gt.astype(jnp.int32), axis=1, keepdims=True)
    need = kq - cnt_gt
    eq = key == res
    cnt_eq = jnp.sum(eq.astype(jnp.int32), axis=1, keepdims=True)

    resc_ref[...] = jnp.full((tq, 1), s - 1, jnp.int32)

    @pl.when(jnp.any(cnt_eq > need))
    def _tie_path():
        needm1 = need - 1
        resc = jnp.zeros((tq, 1), jnp.int32)
        for bit in range((s - 1).bit_length() - 1, -1, -1):
            candc = resc | jnp.int32(1 << bit)
            cnte = jnp.sum(
                (eq & (j < candc)).astype(jnp.int32), axis=1, keepdims=True
            )
            resc = jnp.where(cnte <= needm1, candc, resc)
        resc_ref[...] = resc

    sel = eq & (j <= resc_ref[...]) & (need > 0)
    out_ref[0] = gt | sel


def kernel(x, index_scores, training_step):
    B, S, _ = index_scores.shape
    progress_static = min(1.0, _STEP_CONST / _MAX_STEPS)
    k_static = min(int(_START_K + (_END_K - _START_K) * progress_static), S)
    TQ = 256
    mask = pl.pallas_call(
        functools.partial(_mask_kernel, k_static=k_static, tq=TQ, s=S),
        grid=(B, S // TQ),
        in_specs=[pl.BlockSpec((1, TQ, S), lambda b, i: (b, i, 0))],
        out_specs=pl.BlockSpec((1, TQ, S), lambda b, i: (b, i, 0)),
        out_shape=jax.ShapeDtypeStruct((B, S, S), jnp.bool_),
        scratch_shapes=[pltpu.VMEM((TQ, 1), jnp.int32)],
        compiler_params=pltpu.CompilerParams(
            dimension_semantics=("parallel", "parallel")
        ),
    )(index_scores)

    progress_traced = jnp.minimum(1.0, training_step / _MAX_STEPS)
    k_traced = _START_K + (_END_K - _START_K) * progress_traced
    k_val = jnp.minimum(k_traced.astype(jnp.int32), S)
    k_values = jnp.broadcast_to(k_val, (B, S)).astype(jnp.int32)
    return (mask, k_values)


# speedup vs baseline: 21.8930x; 1.0035x over previous
import functools

import jax
import jax.numpy as jnp
from jax.experimental import pallas as pl
from jax.experimental.pallas import tpu as pltpu

_START_K = 16
_END_K = 256
_MAX_STEPS = 1000
_STEP_CONST = 500

_INT_MIN = -(2**31)


def _mask_kernel(scores_ref, out_ref, resc_ref, *, k_static, tq, s):
    blk_q = pl.program_id(1)
    scores = scores_ref[0]
    bits = jax.lax.bitcast_convert_type(scores, jnp.int32)
    key = bits ^ (jax.lax.shift_right_arithmetic(bits, 31) & jnp.int32(0x7FFFFFFF))
    q = blk_q * tq + jax.lax.broadcasted_iota(jnp.int32, (tq, 1), 0)
    j = jax.lax.broadcasted_iota(jnp.int32, (tq, s), 1)
    key = jnp.where(j <= q, key, jnp.int32(_INT_MIN))
    kq = jnp.minimum(jnp.int32(k_static), q + 1)

    hi = jax.lax.shift_right_arithmetic(key, 16).astype(jnp.int16)
    lo = (key ^ jnp.int32(0x8000)).astype(jnp.int16)

    def cnt16(mask):
        a = mask.astype(jnp.int16)
        w = s
        while w > 128:
            w //= 2
            a = a[:, :w] + a[:, w:]
        return jnp.sum(a.astype(jnp.int32), axis=1, keepdims=True)

    cnt = cnt16(hi >= 0)
    res_hi = jnp.where(cnt >= kq, jnp.int32(0), jnp.int32(-(2**15)))
    for bit in range(14, -1, -1):
        cand = res_hi | jnp.int32(1 << bit)
        cnt = cnt16(hi >= cand.astype(jnp.int16))
        res_hi = jnp.where(cnt >= kq, cand, res_hi)

    res_hi16 = res_hi.astype(jnp.int16)
    ehi = hi == res_hi16
    cnt_hi_gt = cnt16(hi > res_hi16)

    kq_lo = kq - cnt_hi_gt
    cnt = cnt16(ehi & (lo >= 0))
    res_lo = jnp.where(cnt >= kq_lo, jnp.int32(0), jnp.int32(-(2**15)))
    for bit in range(14, -1, -1):
        cand = res_lo | jnp.int32(1 << bit)
        cnt = cnt16(ehi & (lo >= cand.astype(jnp.int16)))
        res_lo = jnp.where(cnt >= kq_lo, cand, res_lo)

    res_lo16 = res_lo.astype(jnp.int16)
    cnt_gt = cnt_hi_gt + cnt16(ehi & (lo > res_lo16))
    need = kq - cnt_gt
    cnt_eq = cnt16(ehi & (lo == res_lo16))

    res = (res_hi << 16) | ((res_lo ^ jnp.int32(0x8000)) & jnp.int32(0xFFFF))
    gt = key > res
    eq = key == res

    resc_ref[...] = jnp.full((tq, 1), s - 1, jnp.int32)

    @pl.when(jnp.any(cnt_eq > need))
    def _tie_path():
        needm1 = need - 1
        resc = jnp.zeros((tq, 1), jnp.int32)
        for bit in range((s - 1).bit_length() - 1, -1, -1):
            candc = resc | jnp.int32(1 << bit)
            cnte = jnp.sum(
                (eq & (j < candc)).astype(jnp.int32), axis=1, keepdims=True
            )
            resc = jnp.where(cnte <= needm1, candc, resc)
        resc_ref[...] = resc

    sel = eq & (j <= resc_ref[...]) & (need > 0)
    out_ref[0] = gt | sel


def kernel(x, index_scores, training_step):
    B, S, _ = index_scores.shape
    progress_static = min(1.0, _STEP_CONST / _MAX_STEPS)
    k_static = min(int(_START_K + (_END_K - _START_K) * progress_static), S)
    TQ = 256
    mask = pl.pallas_call(
        functools.partial(_mask_kernel, k_static=k_static, tq=TQ, s=S),
        grid=(B, S // TQ),
        in_specs=[pl.BlockSpec((1, TQ, S), lambda b, i: (b, i, 0))],
        out_specs=pl.BlockSpec((1, TQ, S), lambda b, i: (b, i, 0)),
        out_shape=jax.ShapeDtypeStruct((B, S, S), jnp.bool_),
        scratch_shapes=[pltpu.VMEM((TQ, 1), jnp.int32)],
        compiler_params=pltpu.CompilerParams(
            dimension_semantics=("parallel", "parallel")
        ),
    )(index_scores)

    progress_traced = jnp.minimum(1.0, training_step / _MAX_STEPS)
    k_traced = _START_K + (_END_K - _START_K) * progress_traced
    k_val = jnp.minimum(k_traced.astype(jnp.int32), S)
    k_values = jnp.broadcast_to(k_val, (B, S)).astype(jnp.int32)
    return (mask, k_values)


# speedup vs baseline: 23.4687x; 1.0720x over previous
import functools

import jax
import jax.numpy as jnp
from jax.experimental import pallas as pl
from jax.experimental.pallas import tpu as pltpu

_START_K = 16
_END_K = 256
_MAX_STEPS = 1000
_STEP_CONST = 500

_INT_MIN = -(2**31)


def _mask_kernel(scores_ref, out_ref, resc_ref, *, k_static, tq, s):
    blk_q = pl.program_id(1)
    scores = scores_ref[0]
    bits = jax.lax.bitcast_convert_type(scores, jnp.int32)
    key = bits ^ (jax.lax.shift_right_arithmetic(bits, 31) & jnp.int32(0x7FFFFFFF))
    q = blk_q * tq + jax.lax.broadcasted_iota(jnp.int32, (tq, 1), 0)
    j = jax.lax.broadcasted_iota(jnp.int32, (tq, s), 1)
    key = jnp.where(j <= q, key, jnp.int32(_INT_MIN))
    kq = jnp.minimum(jnp.int32(k_static), q + 1)

    hi = jax.lax.shift_right_arithmetic(key, 16).astype(jnp.int16)
    lo = (key ^ jnp.int32(0x8000)).astype(jnp.int16)

    def cnt16(mask):
        a = mask.astype(jnp.int16)
        w = s
        while w > 128:
            w //= 2
            a = a[:, :w] + a[:, w:]
        return jnp.sum(a.astype(jnp.int32), axis=1, keepdims=True)

    cnt = cnt16(hi >= 0)
    res_hi = jnp.where(cnt >= kq, jnp.int32(0), jnp.int32(-(2**15)))
    for bit in range(14, -1, -1):
        cand = res_hi | jnp.int32(1 << bit)
        cnt = cnt16(hi >= cand.astype(jnp.int16))
        res_hi = jnp.where(cnt >= kq, cand, res_hi)

    res_hi16 = res_hi.astype(jnp.int16)
    ehi = hi == res_hi16
    cnt_hi_gt = cnt16(hi > res_hi16)

    kq_lo = kq - cnt_hi_gt
    loe = jnp.where(ehi, lo, jnp.int16(-(2**15)))
    cnt = cnt16(loe >= 0)
    res_lo = jnp.where(cnt >= kq_lo, jnp.int32(0), jnp.int32(-(2**15)))
    for bit in range(14, -1, -1):
        cand = res_lo | jnp.int32(1 << bit)
        cnt = cnt16(loe >= cand.astype(jnp.int16))
        res_lo = jnp.where(cnt >= kq_lo, cand, res_lo)

    res_lo16 = res_lo.astype(jnp.int16)
    cnt_gt = cnt_hi_gt + cnt16(loe > res_lo16)
    need = kq - cnt_gt
    cnt_eq = cnt16(ehi & (lo == res_lo16))

    res = (res_hi << 16) | ((res_lo ^ jnp.int32(0x8000)) & jnp.int32(0xFFFF))
    gt = key > res
    eq = key == res

    resc_ref[...] = jnp.full((tq, 1), s - 1, jnp.int32)

    @pl.when(jnp.any(cnt_eq > need))
    def _tie_path():
        needm1 = need - 1
        resc = jnp.zeros((tq, 1), jnp.int32)
        for bit in range((s - 1).bit_length() - 1, -1, -1):
            candc = resc | jnp.int32(1 << bit)
            cnte = jnp.sum(
                (eq & (j < candc)).astype(jnp.int32), axis=1, keepdims=True
            )
            resc = jnp.where(cnte <= needm1, candc, resc)
        resc_ref[...] = resc

    sel = eq & (j <= resc_ref[...]) & (need > 0)
    out_ref[0] = gt | sel


def kernel(x, index_scores, training_step):
    B, S, _ = index_scores.shape
    progress_static = min(1.0, _STEP_CONST / _MAX_STEPS)
    k_static = min(int(_START_K + (_END_K - _START_K) * progress_static), S)
    TQ = 256
    mask = pl.pallas_call(
        functools.partial(_mask_kernel, k_static=k_static, tq=TQ, s=S),
        grid=(B, S // TQ),
        in_specs=[pl.BlockSpec((1, TQ, S), lambda b, i: (b, i, 0))],
        out_specs=pl.BlockSpec((1, TQ, S), lambda b, i: (b, i, 0)),
        out_shape=jax.ShapeDtypeStruct((B, S, S), jnp.bool_),
        scratch_shapes=[pltpu.VMEM((TQ, 1), jnp.int32)],
        compiler_params=pltpu.CompilerParams(
            dimension_semantics=("parallel", "parallel")
        ),
    )(index_scores)

    progress_traced = jnp.minimum(1.0, training_step / _MAX_STEPS)
    k_traced = _START_K + (_END_K - _START_K) * progress_traced
    k_val = jnp.minimum(k_traced.astype(jnp.int32), S)
    k_values = jnp.broadcast_to(k_val, (B, S)).astype(jnp.int32)
    return (mask, k_values)


# speedup vs baseline: 25.8552x; 1.1017x over previous
import functools

import jax
import jax.numpy as jnp
from jax.experimental import pallas as pl
from jax.experimental.pallas import tpu as pltpu

_START_K = 16
_END_K = 256
_MAX_STEPS = 1000
_STEP_CONST = 500

_INT_MIN = -(2**31)


def _mask_kernel(scores_ref, out_ref, *, k_static, tq, s):
    blk_q = pl.program_id(1)
    scores = scores_ref[0]
    bits = jax.lax.bitcast_convert_type(scores, jnp.int32)
    key = bits ^ (jax.lax.shift_right_arithmetic(bits, 31) & jnp.int32(0x7FFFFFFF))
    q = blk_q * tq + jax.lax.broadcasted_iota(jnp.int32, (tq, 1), 0)
    j = jax.lax.broadcasted_iota(jnp.int32, (tq, s), 1)
    key = jnp.where(j <= q, key, jnp.int32(_INT_MIN))
    kq = jnp.minimum(jnp.int32(k_static), q + 1)

    hi = jax.lax.shift_right_arithmetic(key, 16).astype(jnp.int16)
    lo = (key ^ jnp.int32(0x8000)).astype(jnp.int16)

    def cnt16(mask):
        a = mask.astype(jnp.int16)
        w = s
        while w > 128:
            w //= 2
            a = a[:, :w] + a[:, w:]
        return jnp.sum(a.astype(jnp.int32), axis=1, keepdims=True)

    cnt = cnt16(hi >= 0)
    res_hi = jnp.where(cnt >= kq, jnp.int32(0), jnp.int32(-(2**15)))
    for bit in range(14, -1, -1):
        cand = res_hi | jnp.int32(1 << bit)
        cnt = cnt16(hi >= cand.astype(jnp.int16))
        res_hi = jnp.where(cnt >= kq, cand, res_hi)

    res_hi16 = res_hi.astype(jnp.int16)
    ehi = hi == res_hi16
    cnt_hi_gt = cnt16(hi > res_hi16)

    kq_lo = kq - cnt_hi_gt
    loe = jnp.where(ehi, lo, jnp.int16(-(2**15)))
    cnt = cnt16(loe >= 0)
    res_lo = jnp.where(cnt >= kq_lo, jnp.int32(0), jnp.int32(-(2**15)))
    for bit in range(14, -1, -1):
        cand = res_lo | jnp.int32(1 << bit)
        cnt = cnt16(loe >= cand.astype(jnp.int16))
        res_lo = jnp.where(cnt >= kq_lo, cand, res_lo)

    res_lo16 = res_lo.astype(jnp.int16)
    cnt_gt = cnt_hi_gt + cnt16(loe > res_lo16)
    need = kq - cnt_gt
    cnt_eq = cnt16(ehi & (lo == res_lo16))

    res = (res_hi << 16) | ((res_lo ^ jnp.int32(0x8000)) & jnp.int32(0xFFFF))
    gt = key > res
    eq = key == res

    has_ties = jnp.any(cnt_eq > need)

    @pl.when(has_ties)
    def _tie_path():
        needm1 = need - 1
        resc = jnp.zeros((tq, 1), jnp.int32)
        for bit in range((s - 1).bit_length() - 1, -1, -1):
            candc = resc | jnp.int32(1 << bit)
            cnte = jnp.sum(
                (eq & (j < candc)).astype(jnp.int32), axis=1, keepdims=True
            )
            resc = jnp.where(cnte <= needm1, candc, resc)
        out_ref[0] = gt | (eq & (j <= resc) & (need > 0))

    @pl.when(jnp.logical_not(has_ties))
    def _simple_path():
        out_ref[0] = gt | eq


def kernel(x, index_scores, training_step):
    B, S, _ = index_scores.shape
    progress_static = min(1.0, _STEP_CONST / _MAX_STEPS)
    k_static = min(int(_START_K + (_END_K - _START_K) * progress_static), S)
    TQ = 256
    mask = pl.pallas_call(
        functools.partial(_mask_kernel, k_static=k_static, tq=TQ, s=S),
        grid=(B, S // TQ),
        in_specs=[pl.BlockSpec((1, TQ, S), lambda b, i: (b, i, 0))],
        out_specs=pl.BlockSpec((1, TQ, S), lambda b, i: (b, i, 0)),
        out_shape=jax.ShapeDtypeStruct((B, S, S), jnp.bool_),
        compiler_params=pltpu.CompilerParams(
            dimension_semantics=("parallel", "parallel")
        ),
    )(index_scores)

    progress_traced = jnp.minimum(1.0, training_step / _MAX_STEPS)
    k_traced = _START_K + (_END_K - _START_K) * progress_traced
    k_val = jnp.minimum(k_traced.astype(jnp.int32), S)
    k_values = jnp.broadcast_to(k_val, (B, S)).astype(jnp.int32)
    return (mask, k_values)
